# baseline (speedup 1.0000x reference)
import functools

import jax
import jax.numpy as jnp
from jax import lax
from jax.experimental import pallas as pl
from jax.experimental.pallas import tpu as pltpu
from jax.experimental.pallas import tpu_sc as plsc

H = 256
HH = 128
NNODES = 10000
NEDGES = 160000
NS = 16
NC = 2
EC = 80
EPT = NEDGES // NS
NCHUNK = EPT // EC
NPAD = 10240
RPT = NPAD // NS
CG = HH // 16

_mesh = plsc.VectorSubcoreMesh(core_axis_name="c", subcore_axis_name="s")


@functools.partial(
    pl.kernel,
    out_type=jax.ShapeDtypeStruct((NC, NPAD, HH), jnp.float32),
    mesh=_mesh,
    scratch_types=[
        pltpu.VMEM((EPT,), jnp.int32),
        pltpu.VMEM((3, EC), jnp.int32),
        pltpu.VMEM((EC, HH), jnp.float32),
        pltpu.VMEM((EC, HH), jnp.float32),
        pltpu.VMEM((EC, HH), jnp.float32),
        pltpu.VMEM_SHARED((NPAD, HH), jnp.float32),
        pltpu.SemaphoreType.DMA,
        pltpu.SemaphoreType.DMA,
        pltpu.SemaphoreType.DMA,
        pltpu.SemaphoreType.DMA,
        pltpu.SemaphoreType.DMA,
        pltpu.SemaphoreType.DMA,
        pltpu.SemaphoreType.DMA,
        pltpu.SemaphoreType.DMA,
        pltpu.SemaphoreType.DMA,
        pltpu.SemaphoreType.DMA,
        pltpu.SemaphoreType.DMA,
        pltpu.SemaphoreType.DMA,
    ],
)
def _edge_kernel(src_hbm, dst_hbm, x2_hbm, ea3_hbm, out_hbm,
                 gxa, didx, er0, er1, er2, aggr,
                 smx0, smx1, smx2, sme0, sme1, sme2,
                 smd0, smd1, smd2, sms0, sms1, sms2):
    cid = lax.axis_index("c")
    sid = lax.axis_index("s")
    er = (er0, er1, er2)
    semx = (smx0, smx1, smx2)
    seme = (sme0, sme1, sme2)
    semd = (smd0, smd1, smd2)
    sems = (sms0, sms1, sms2)
    te0 = sid * EPT
    r0 = sid * RPT

    pltpu.sync_copy(src_hbm.at[pl.ds(te0, EPT)], gxa)
    iota16 = lax.iota(jnp.int32, 16)

    @plsc.parallel_loop(0, EPT // 16, unroll=4)
    def _bld(g):
        o = g * 16
        gxa[pl.ds(o, 16)] = gxa[pl.ds(o, 16)] * 2 + cid

    zero16 = jnp.zeros((16,), jnp.float32)

    @plsc.parallel_loop(0, EC, unroll=2)
    def _zrow(r):
        for g in range(CG):
            er0[r, pl.ds(g * 16, 16)] = zero16

    for j in range(RPT // EC):
        pltpu.sync_copy(er0.at[pl.ds(0, EC)], aggr.at[pl.ds(r0 + j * EC, EC)])

    def issue_a(k, slot):
        o = k * EC
        pltpu.async_copy(ea3_hbm.at[pl.ds(te0 + o, EC), cid], er[slot],
                         seme[slot])
        pltpu.async_copy(dst_hbm.at[pl.ds(te0 + o, EC)], didx.at[slot],
                         semd[slot])

    def wait_a(slot):
        pltpu.make_async_copy(ea3_hbm.at[pl.ds(te0, EC), cid], er[slot],
                              seme[slot]).wait()
        pltpu.make_async_copy(dst_hbm.at[pl.ds(te0, EC)], didx.at[slot],
                              semd[slot]).wait()

    def issue_b(k, slot):
        o = k * EC
        pltpu.async_copy(x2_hbm.at[gxa.at[pl.ds(o, EC)]], er[slot],
                         semx[slot], add=True)

    def wait_b(slot):
        pltpu.make_async_copy(x2_hbm.at[gxa.at[pl.ds(0, EC)]], er[slot],
                              semx[slot]).wait()

    def relu(slot):
        ers = er[slot]

        @plsc.parallel_loop(0, EC, unroll=2)
        def _row(r):
            for g in range(CG):
                v = ers[r, pl.ds(g * 16, 16)]
                ers[r, pl.ds(g * 16, 16)] = jnp.maximum(v, 0.0)

    def issue_s(slot):
        pltpu.async_copy(er[slot], aggr.at[didx.at[slot]], sems[slot],
                         add=True)

    def wait_s(slot):
        pltpu.make_async_copy(er[slot], aggr.at[didx.at[slot]],
                              sems[slot]).wait()

    def body(k, s, s1, s2, first=False, last=False, next_a=True):
        if not first:
            wait_s(s2)
        if not last and next_a:
            issue_a(k + 2, s2)
        wait_b(s)
        relu(s)
        issue_s(s)
        if not last:
            wait_a(s1)
            issue_b(k + 1, s1)

    issue_a(0, 0)
    wait_a(0)
    issue_b(0, 0)
    issue_a(1, 1)
    plsc.subcore_barrier()

    body(0, 0, 1, 2, first=True)

    def triple(k3, _):
        k = k3 * 3 + 1
        body(k, 1, 2, 0)
        body(k + 1, 2, 0, 1)
        body(k + 2, 0, 1, 2)
        return 0

    lax.fori_loop(0, (NCHUNK - 5) // 3, triple, 0)
    body(121, 1, 2, 0)
    body(122, 2, 0, 1)
    body(123, 0, 1, 2, next_a=False)
    body(124, 1, 2, 0, last=True)
    wait_s(1)

    plsc.subcore_barrier()

    pltpu.sync_copy(aggr.at[pl.ds(r0, RPT)], out_hbm.at[cid, pl.ds(r0, RPT)])


BM = 2000


def _mlp_body(x_ref, a_ref, w1_ref, b1_ref, w2_ref, b2_ref, o_ref):
    a = jnp.concatenate([a_ref[0], a_ref[1]], axis=-1)
    h = x_ref[...] + a
    h1 = jnp.dot(h, w1_ref[...], preferred_element_type=jnp.float32)
    h1 = jnp.maximum(h1 + b1_ref[...], 0.0)
    o_ref[...] = (jnp.dot(h1, w2_ref[...], preferred_element_type=jnp.float32)
                  + b2_ref[...])


def _mlp(x, aggr2, W1, b1, W2, b2):
    return pl.pallas_call(
        _mlp_body,
        grid=(NNODES // BM,),
        in_specs=[
            pl.BlockSpec((BM, H), lambda i: (i, 0)),
            pl.BlockSpec((NC, BM, HH), lambda i: (0, i, 0)),
            pl.BlockSpec((H, H), lambda i: (0, 0)),
            pl.BlockSpec((1, H), lambda i: (0, 0)),
            pl.BlockSpec((H, H), lambda i: (0, 0)),
            pl.BlockSpec((1, H), lambda i: (0, 0)),
        ],
        out_specs=pl.BlockSpec((BM, H), lambda i: (i, 0)),
        out_shape=jax.ShapeDtypeStruct((NNODES, H), jnp.float32),
    )(x, aggr2, W1, b1.reshape(1, H), W2, b2.reshape(1, H))


def kernel(x, edge_index, edge_attr, W1, b1, W2, b2):
    ei = edge_index.astype(jnp.int32)
    src = ei[0]
    dst = ei[1]
    x2 = x.reshape(NC * NNODES, HH)
    ea3 = edge_attr.reshape(NEDGES, NC, HH)
    aggr2 = _edge_kernel(src, dst, x2, ea3)
    return _mlp(x, aggr2, W1, b1, W2, b2)


# speedup vs baseline: 1.0653x; 1.0653x over previous
import functools

import jax
import jax.numpy as jnp
from jax import lax
from jax.experimental import pallas as pl
from jax.experimental.pallas import tpu as pltpu
from jax.experimental.pallas import tpu_sc as plsc

H = 256
HH = 128
NNODES = 10000
NEDGES = 160000
NS = 16
NC = 2
EC = 80
EPT = NEDGES // NS
NCHUNK = EPT // EC
NPAD = 10240
RPT = NPAD // NS
CG = HH // 16

_mesh = plsc.VectorSubcoreMesh(core_axis_name="c", subcore_axis_name="s")


@functools.partial(
    pl.kernel,
    out_type=jax.ShapeDtypeStruct((NC, NPAD, HH), jnp.float32),
    mesh=_mesh,
    scratch_types=[
        pltpu.VMEM((EPT,), jnp.int32),
        pltpu.VMEM((3, EC), jnp.int32),
        pltpu.VMEM((3, EC), jnp.int32),
        pltpu.VMEM((EC, HH), jnp.float32),
        pltpu.VMEM((EC, HH), jnp.float32),
        pltpu.VMEM((EC, HH), jnp.float32),
        pltpu.VMEM_SHARED((NPAD, HH), jnp.float32),
        pltpu.SemaphoreType.DMA,
        pltpu.SemaphoreType.DMA,
        pltpu.SemaphoreType.DMA,
        pltpu.SemaphoreType.DMA,
        pltpu.SemaphoreType.DMA,
        pltpu.SemaphoreType.DMA,
        pltpu.SemaphoreType.DMA,
        pltpu.SemaphoreType.DMA,
        pltpu.SemaphoreType.DMA,
        pltpu.SemaphoreType.DMA,
        pltpu.SemaphoreType.DMA,
        pltpu.SemaphoreType.DMA,
    ],
)
def _edge_kernel(src_hbm, dst_hbm, x2_hbm, ea2_hbm, out_hbm,
                 gxa, geb, didx, er0, er1, er2, aggr,
                 smx0, smx1, smx2, sme0, sme1, sme2,
                 smd0, smd1, smd2, sms0, sms1, sms2):
    cid = lax.axis_index("c")
    sid = lax.axis_index("s")
    er = (er0, er1, er2)
    semx = (smx0, smx1, smx2)
    seme = (sme0, sme1, sme2)
    semd = (smd0, smd1, smd2)
    sems = (sms0, sms1, sms2)
    te0 = sid * EPT
    r0 = sid * RPT

    pltpu.sync_copy(src_hbm.at[pl.ds(te0, EPT)], gxa)
    iota16 = lax.iota(jnp.int32, 16)

    @plsc.parallel_loop(0, EPT // 16, unroll=4)
    def _bld(g):
        o = g * 16
        gxa[pl.ds(o, 16)] = gxa[pl.ds(o, 16)] * 2 + cid

    zero16 = jnp.zeros((16,), jnp.float32)

    @plsc.parallel_loop(0, EC, unroll=2)
    def _zrow(r):
        for g in range(CG):
            er0[r, pl.ds(g * 16, 16)] = zero16

    for j in range(RPT // EC):
        pltpu.sync_copy(er0.at[pl.ds(0, EC)], aggr.at[pl.ds(r0 + j * EC, EC)])

    def issue_a(k, slot):
        o = k * EC
        base = (te0 + o) * 2 + cid
        for g in range(EC // 16):
            geb[slot, pl.ds(g * 16, 16)] = iota16 * 2 + (base + g * 32)
        pltpu.async_copy(ea2_hbm.at[geb.at[slot]], er[slot], seme[slot])
        pltpu.async_copy(dst_hbm.at[pl.ds(te0 + o, EC)], didx.at[slot],
                         semd[slot])

    def wait_a(slot):
        pltpu.make_async_copy(ea2_hbm.at[geb.at[slot]], er[slot],
                              seme[slot]).wait()
        pltpu.make_async_copy(dst_hbm.at[pl.ds(te0, EC)], didx.at[slot],
                              semd[slot]).wait()

    def issue_b(k, slot):
        o = k * EC
        pltpu.async_copy(x2_hbm.at[gxa.at[pl.ds(o, EC)]], er[slot],
                         semx[slot], add=True)

    def wait_b(slot):
        pltpu.make_async_copy(x2_hbm.at[gxa.at[pl.ds(0, EC)]], er[slot],
                              semx[slot]).wait()

    def relu(slot):
        ers = er[slot]

        @plsc.parallel_loop(0, EC, unroll=4)
        def _row(r):
            for g in range(CG):
                v = ers[r, pl.ds(g * 16, 16)]
                ers[r, pl.ds(g * 16, 16)] = jnp.maximum(v, 0.0)

    def issue_s(slot):
        pltpu.async_copy(er[slot], aggr.at[didx.at[slot]], sems[slot],
                         add=True)

    def wait_s(slot):
        pltpu.make_async_copy(er[slot], aggr.at[didx.at[slot]],
                              sems[slot]).wait()

    def body(k, s, s1, s2, first=False, last=False, next_a=True):
        if not first:
            wait_s(s2)
        if not last and next_a:
            issue_a(k + 2, s2)
        wait_b(s)
        relu(s)
        issue_s(s)
        if not last:
            wait_a(s1)
            issue_b(k + 1, s1)

    issue_a(0, 0)
    wait_a(0)
    issue_b(0, 0)
    issue_a(1, 1)
    plsc.subcore_barrier()

    body(0, 0, 1, 2, first=True)

    def triple(k3, _):
        k = k3 * 3 + 1
        body(k, 1, 2, 0)
        body(k + 1, 2, 0, 1)
        body(k + 2, 0, 1, 2)
        return 0

    lax.fori_loop(0, (NCHUNK - 5) // 3, triple, 0)
    body(121, 1, 2, 0)
    body(122, 2, 0, 1)
    body(123, 0, 1, 2, next_a=False)
    body(124, 1, 2, 0, last=True)
    wait_s(1)

    plsc.subcore_barrier()

    pltpu.sync_copy(aggr.at[pl.ds(r0, RPT)], out_hbm.at[cid, pl.ds(r0, RPT)])


BM = 2000


def _mlp_body(x_ref, a_ref, w1_ref, b1_ref, w2_ref, b2_ref, o_ref):
    a = jnp.concatenate([a_ref[0], a_ref[1]], axis=-1)
    h = x_ref[...] + a
    h1 = jnp.dot(h, w1_ref[...], preferred_element_type=jnp.float32)
    h1 = jnp.maximum(h1 + b1_ref[...], 0.0)
    o_ref[...] = (jnp.dot(h1, w2_ref[...], preferred_element_type=jnp.float32)
                  + b2_ref[...])


def _mlp(x, aggr2, W1, b1, W2, b2):
    return pl.pallas_call(
        _mlp_body,
        grid=(NNODES // BM,),
        in_specs=[
            pl.BlockSpec((BM, H), lambda i: (i, 0)),
            pl.BlockSpec((NC, BM, HH), lambda i: (0, i, 0)),
            pl.BlockSpec((H, H), lambda i: (0, 0)),
            pl.BlockSpec((1, H), lambda i: (0, 0)),
            pl.BlockSpec((H, H), lambda i: (0, 0)),
            pl.BlockSpec((1, H), lambda i: (0, 0)),
        ],
        out_specs=pl.BlockSpec((BM, H), lambda i: (i, 0)),
        out_shape=jax.ShapeDtypeStruct((NNODES, H), jnp.float32),
    )(x, aggr2, W1, b1.reshape(1, H), W2, b2.reshape(1, H))


def kernel(x, edge_index, edge_attr, W1, b1, W2, b2):
    ei = edge_index.astype(jnp.int32)
    src = ei[0]
    dst = ei[1]
    x2 = x.reshape(NC * NNODES, HH)
    ea2 = edge_attr.reshape(NC * NEDGES, HH)
    aggr2 = _edge_kernel(src, dst, x2, ea2)
    return _mlp(x, aggr2, W1, b1, W2, b2)


# speedup vs baseline: 1.1374x; 1.0677x over previous
import functools

import jax
import jax.numpy as jnp
from jax import lax
from jax.experimental import pallas as pl
from jax.experimental.pallas import tpu as pltpu
from jax.experimental.pallas import tpu_sc as plsc

H = 256
HH = 128
NNODES = 10000
NEDGES = 160000
NS = 16
NC = 2
EC = 80
EPT = NEDGES // NS
NCHUNK = EPT // EC
NPAD = 10240
RPT = NPAD // NS
CG = HH // 16

_mesh = plsc.VectorSubcoreMesh(core_axis_name="c", subcore_axis_name="s")


@functools.partial(
    pl.kernel,
    out_type=jax.ShapeDtypeStruct((NC, NPAD, HH), jnp.float32),
    mesh=_mesh,
    scratch_types=[
        pltpu.VMEM((EPT,), jnp.int32),
        pltpu.VMEM((3, EC), jnp.int32),
        pltpu.VMEM((3, EC), jnp.int32),
        pltpu.VMEM((EC, HH), jnp.float32),
        pltpu.VMEM((EC, HH), jnp.float32),
        pltpu.VMEM((EC, HH), jnp.float32),
        pltpu.VMEM_SHARED((NPAD, HH), jnp.float32),
        pltpu.SemaphoreType.DMA,
        pltpu.SemaphoreType.DMA,
        pltpu.SemaphoreType.DMA,
        pltpu.SemaphoreType.DMA,
        pltpu.SemaphoreType.DMA,
        pltpu.SemaphoreType.DMA,
        pltpu.SemaphoreType.DMA,
        pltpu.SemaphoreType.DMA,
        pltpu.SemaphoreType.DMA,
        pltpu.SemaphoreType.DMA,
        pltpu.SemaphoreType.DMA,
        pltpu.SemaphoreType.DMA,
    ],
)
def _edge_kernel(src_hbm, dst_hbm, x2_hbm, ea2_hbm, out_hbm,
                 gxa, geb, didx, er0, er1, er2, aggr,
                 smx0, smx1, smx2, sme0, sme1, sme2,
                 smd0, smd1, smd2, sms0, sms1, sms2):
    cid = lax.axis_index("c")
    sid = lax.axis_index("s")
    er = (er0, er1, er2)
    semx = (smx0, smx1, smx2)
    seme = (sme0, sme1, sme2)
    semd = (smd0, smd1, smd2)
    sems = (sms0, sms1, sms2)
    te0 = sid * EPT
    r0 = sid * RPT

    pltpu.sync_copy(src_hbm.at[pl.ds(te0, EPT)], gxa)
    iota16 = lax.iota(jnp.int32, 16)

    @plsc.parallel_loop(0, EPT // 16, unroll=4)
    def _bld(g):
        o = g * 16
        gxa[pl.ds(o, 16)] = gxa[pl.ds(o, 16)] * 2 + cid

    zero16 = jnp.zeros((16,), jnp.float32)

    @plsc.parallel_loop(0, EC, unroll=2)
    def _zrow(r):
        for g in range(CG):
            er0[r, pl.ds(g * 16, 16)] = zero16

    for j in range(RPT // EC):
        pltpu.sync_copy(er0.at[pl.ds(0, EC)], aggr.at[pl.ds(r0 + j * EC, EC)])

    def issue_a(k, slot):
        o = k * EC
        base = (te0 + o) * 2 + cid
        for g in range(EC // 16):
            geb[slot, pl.ds(g * 16, 16)] = iota16 * 2 + (base + g * 32)
        pltpu.async_copy(ea2_hbm.at[geb.at[slot]], er[slot], seme[slot])
        pltpu.async_copy(dst_hbm.at[pl.ds(te0 + o, EC)], didx.at[slot],
                         semd[slot])

    def wait_a(slot):
        pltpu.make_async_copy(ea2_hbm.at[geb.at[slot]], er[slot],
                              seme[slot]).wait()
        pltpu.make_async_copy(dst_hbm.at[pl.ds(te0, EC)], didx.at[slot],
                              semd[slot]).wait()

    def issue_b(k, slot):
        o = k * EC
        pltpu.async_copy(x2_hbm.at[gxa.at[pl.ds(o, EC)]], er[slot],
                         semx[slot], add=True)

    def wait_b(slot):
        pltpu.make_async_copy(x2_hbm.at[gxa.at[pl.ds(0, EC)]], er[slot],
                              semx[slot]).wait()

    def relu(slot):
        ers = er[slot]

        @plsc.parallel_loop(0, EC, unroll=4)
        def _row(r):
            for g in range(CG):
                v = ers[r, pl.ds(g * 16, 16)]
                ers[r, pl.ds(g * 16, 16)] = jnp.maximum(v, 0.0)

    def issue_s(slot):
        pltpu.async_copy(er[slot], aggr.at[didx.at[slot]], sems[slot],
                         add=True)

    def wait_s(slot):
        pltpu.make_async_copy(er[slot], aggr.at[didx.at[slot]],
                              sems[slot]).wait()

    def body(k, s, s1, s2, first=False, last=False, next_a=True):
        if not first:
            wait_s(s2)
        if not last and next_a:
            issue_a(k + 2, s2)
        if not last:
            wait_a(s1)
            issue_b(k + 1, s1)
        wait_b(s)
        relu(s)
        issue_s(s)

    issue_a(0, 0)
    wait_a(0)
    issue_b(0, 0)
    issue_a(1, 1)
    plsc.subcore_barrier()

    body(0, 0, 1, 2, first=True)

    def triple(k3, _):
        k = k3 * 3 + 1
        body(k, 1, 2, 0)
        body(k + 1, 2, 0, 1)
        body(k + 2, 0, 1, 2)
        return 0

    lax.fori_loop(0, (NCHUNK - 5) // 3, triple, 0)
    body(121, 1, 2, 0)
    body(122, 2, 0, 1)
    body(123, 0, 1, 2, next_a=False)
    body(124, 1, 2, 0, last=True)
    wait_s(1)

    plsc.subcore_barrier()

    pltpu.sync_copy(aggr.at[pl.ds(r0, RPT)], out_hbm.at[cid, pl.ds(r0, RPT)])


BM = 2000


def _mlp_body(x_ref, a_ref, w1_ref, b1_ref, w2_ref, b2_ref, o_ref):
    a = jnp.concatenate([a_ref[0], a_ref[1]], axis=-1)
    h = x_ref[...] + a
    h1 = jnp.dot(h, w1_ref[...], preferred_element_type=jnp.float32)
    h1 = jnp.maximum(h1 + b1_ref[...], 0.0)
    o_ref[...] = (jnp.dot(h1, w2_ref[...], preferred_element_type=jnp.float32)
                  + b2_ref[...])


def _mlp(x, aggr2, W1, b1, W2, b2):
    return pl.pallas_call(
        _mlp_body,
        grid=(NNODES // BM,),
        in_specs=[
            pl.BlockSpec((BM, H), lambda i: (i, 0)),
            pl.BlockSpec((NC, BM, HH), lambda i: (0, i, 0)),
            pl.BlockSpec((H, H), lambda i: (0, 0)),
            pl.BlockSpec((1, H), lambda i: (0, 0)),
            pl.BlockSpec((H, H), lambda i: (0, 0)),
            pl.BlockSpec((1, H), lambda i: (0, 0)),
        ],
        out_specs=pl.BlockSpec((BM, H), lambda i: (i, 0)),
        out_shape=jax.ShapeDtypeStruct((NNODES, H), jnp.float32),
    )(x, aggr2, W1, b1.reshape(1, H), W2, b2.reshape(1, H))


def kernel(x, edge_index, edge_attr, W1, b1, W2, b2):
    ei = edge_index.astype(jnp.int32)
    src = ei[0]
    dst = ei[1]
    x2 = x.reshape(NC * NNODES, HH)
    ea2 = edge_attr.reshape(NC * NEDGES, HH)
    aggr2 = _edge_kernel(src, dst, x2, ea2)
    return _mlp(x, aggr2, W1, b1, W2, b2)


# speedup vs baseline: 1.1502x; 1.0113x over previous
import functools

import jax
import jax.numpy as jnp
from jax import lax
from jax.experimental import pallas as pl
from jax.experimental.pallas import tpu as pltpu
from jax.experimental.pallas import tpu_sc as plsc

H = 256
HH = 128
NNODES = 10000
NEDGES = 160000
NS = 16
NC = 2
EC = 80
EPT = NEDGES // NS
NCHUNK = EPT // EC
NPAD = 10240
RPT = NPAD // NS
CG = HH // 16

_mesh = plsc.VectorSubcoreMesh(core_axis_name="c", subcore_axis_name="s")


@functools.partial(
    pl.kernel,
    out_type=jax.ShapeDtypeStruct((NC, NPAD, HH), jnp.float32),
    mesh=_mesh,
    scratch_types=[
        pltpu.VMEM((EPT,), jnp.int32),
        pltpu.VMEM((3, EC), jnp.int32),
        pltpu.VMEM((3, EC), jnp.int32),
        pltpu.VMEM((EC, HH), jnp.float32),
        pltpu.VMEM((EC, HH), jnp.float32),
        pltpu.VMEM((EC, HH), jnp.float32),
        pltpu.VMEM_SHARED((NPAD, HH), jnp.float32),
        pltpu.SemaphoreType.DMA,
        pltpu.SemaphoreType.DMA,
        pltpu.SemaphoreType.DMA,
        pltpu.SemaphoreType.DMA,
        pltpu.SemaphoreType.DMA,
        pltpu.SemaphoreType.DMA,
        pltpu.SemaphoreType.DMA,
        pltpu.SemaphoreType.DMA,
        pltpu.SemaphoreType.DMA,
        pltpu.SemaphoreType.DMA,
        pltpu.SemaphoreType.DMA,
        pltpu.SemaphoreType.DMA,
        pltpu.SemaphoreType.DMA,
    ],
)
def _edge_kernel(src_hbm, dst_hbm, x2_hbm, ea2_hbm, out_hbm,
                 gxa, geb, didx, er0, er1, er2, aggr,
                 smx0, smx1, smx2, sme0, sme1, sme2,
                 smd0, smd1, smd2, sms0, sms1, sms2, smz):
    cid = lax.axis_index("c")
    sid = lax.axis_index("s")
    er = (er0, er1, er2)
    semx = (smx0, smx1, smx2)
    seme = (sme0, sme1, sme2)
    semd = (smd0, smd1, smd2)
    sems = (sms0, sms1, sms2)
    te0 = sid * EPT
    r0 = sid * RPT

    gxa_cp = pltpu.async_copy(src_hbm.at[pl.ds(te0, EPT)], gxa, smz)
    iota16 = lax.iota(jnp.int32, 16)
    zero16 = jnp.zeros((16,), jnp.float32)

    @plsc.parallel_loop(0, EC, unroll=2)
    def _zrow(r):
        for g in range(CG):
            er2[r, pl.ds(g * 16, 16)] = zero16

    gxa_cp.wait()

    @plsc.parallel_loop(0, EPT // 16, unroll=4)
    def _bld(g):
        o = g * 16
        gxa[pl.ds(o, 16)] = gxa[pl.ds(o, 16)] * 2 + cid

    for j in range(RPT // EC):
        pltpu.async_copy(er2.at[pl.ds(0, EC)], aggr.at[pl.ds(r0 + j * EC, EC)],
                         smz)

    def issue_a(k, slot):
        o = k * EC
        base = (te0 + o) * 2 + cid
        for g in range(EC // 16):
            geb[slot, pl.ds(g * 16, 16)] = iota16 * 2 + (base + g * 32)
        pltpu.async_copy(ea2_hbm.at[geb.at[slot]], er[slot], seme[slot])
        pltpu.async_copy(dst_hbm.at[pl.ds(te0 + o, EC)], didx.at[slot],
                         semd[slot])

    def wait_a(slot):
        pltpu.make_async_copy(ea2_hbm.at[geb.at[slot]], er[slot],
                              seme[slot]).wait()
        pltpu.make_async_copy(dst_hbm.at[pl.ds(te0, EC)], didx.at[slot],
                              semd[slot]).wait()

    def issue_b(k, slot):
        o = k * EC
        pltpu.async_copy(x2_hbm.at[gxa.at[pl.ds(o, EC)]], er[slot],
                         semx[slot], add=True)

    def wait_b(slot):
        pltpu.make_async_copy(x2_hbm.at[gxa.at[pl.ds(0, EC)]], er[slot],
                              semx[slot]).wait()

    def relu(slot):
        ers = er[slot]

        @plsc.parallel_loop(0, EC, unroll=4)
        def _row(r):
            for g in range(CG):
                v = ers[r, pl.ds(g * 16, 16)]
                ers[r, pl.ds(g * 16, 16)] = jnp.maximum(v, 0.0)

    def issue_s(slot):
        pltpu.async_copy(er[slot], aggr.at[didx.at[slot]], sems[slot],
                         add=True)

    def wait_s(slot):
        pltpu.make_async_copy(er[slot], aggr.at[didx.at[slot]],
                              sems[slot]).wait()

    def body(k, s, s1, s2, first=False, last=False, next_a=True):
        if not first:
            wait_s(s2)
        if not last and next_a:
            issue_a(k + 2, s2)
        if not last:
            wait_a(s1)
            issue_b(k + 1, s1)
        wait_b(s)
        relu(s)
        issue_s(s)

    issue_a(0, 0)
    issue_a(1, 1)
    wait_a(0)
    issue_b(0, 0)
    for j in range(RPT // EC):
        pltpu.make_async_copy(er2.at[pl.ds(0, EC)],
                              aggr.at[pl.ds(r0 + j * EC, EC)], smz).wait()
    plsc.subcore_barrier()

    body(0, 0, 1, 2, first=True)

    def triple(k3, _):
        k = k3 * 3 + 1
        body(k, 1, 2, 0)
        body(k + 1, 2, 0, 1)
        body(k + 2, 0, 1, 2)
        return 0

    lax.fori_loop(0, (NCHUNK - 5) // 3, triple, 0)
    body(121, 1, 2, 0)
    body(122, 2, 0, 1)
    body(123, 0, 1, 2, next_a=False)
    body(124, 1, 2, 0, last=True)
    wait_s(1)

    plsc.subcore_barrier()

    pltpu.sync_copy(aggr.at[pl.ds(r0, RPT)], out_hbm.at[cid, pl.ds(r0, RPT)])


BM = 2000


def _mlp_body(x_ref, a_ref, w1_ref, b1_ref, w2_ref, b2_ref, o_ref):
    a = jnp.concatenate([a_ref[0], a_ref[1]], axis=-1)
    h = x_ref[...] + a
    h1 = jnp.dot(h, w1_ref[...], preferred_element_type=jnp.float32)
    h1 = jnp.maximum(h1 + b1_ref[...], 0.0)
    o_ref[...] = (jnp.dot(h1, w2_ref[...], preferred_element_type=jnp.float32)
                  + b2_ref[...])


def _mlp(x, aggr2, W1, b1, W2, b2):
    return pl.pallas_call(
        _mlp_body,
        grid=(NNODES // BM,),
        in_specs=[
            pl.BlockSpec((BM, H), lambda i: (i, 0)),
            pl.BlockSpec((NC, BM, HH), lambda i: (0, i, 0)),
            pl.BlockSpec((H, H), lambda i: (0, 0)),
            pl.BlockSpec((1, H), lambda i: (0, 0)),
            pl.BlockSpec((H, H), lambda i: (0, 0)),
            pl.BlockSpec((1, H), lambda i: (0, 0)),
        ],
        out_specs=pl.BlockSpec((BM, H), lambda i: (i, 0)),
        out_shape=jax.ShapeDtypeStruct((NNODES, H), jnp.float32),
    )(x, aggr2, W1, b1.reshape(1, H), W2, b2.reshape(1, H))


def kernel(x, edge_index, edge_attr, W1, b1, W2, b2):
    ei = edge_index.astype(jnp.int32)
    src = ei[0]
    dst = ei[1]
    x2 = x.reshape(NC * NNODES, HH)
    ea2 = edge_attr.reshape(NC * NEDGES, HH)
    aggr2 = _edge_kernel(src, dst, x2, ea2)
    return _mlp(x, aggr2, W1, b1, W2, b2)
